# baseline (speedup 1.0000x reference)
import functools

import jax
import jax.numpy as jnp
from jax import lax
from jax.experimental import pallas as pl
from jax.experimental.pallas import tpu as pltpu
from jax.experimental.pallas import tpu_sc as plsc

S = 2048
D = 1024
FF = 512
E = 64
BM = 128
R = 10240
G = 80
TB = 256
NTILES = 32


def _router_body(x_ref, wr_ref, moe_ref, ids_ref, rank_ref, cnt_ref, acc):
    pid = pl.program_id(0)
    x = x_ref[...]
    logits = jnp.dot(x, wr_ref[...], preferred_element_type=jnp.float32)
    m = jnp.max(logits, axis=1, keepdims=True)
    lane = lax.broadcasted_iota(jnp.int32, (TB, E), 1)
    ids = jnp.min(jnp.where(logits == m, lane, E), axis=1, keepdims=True)
    moe_ref[...] = x * jax.nn.sigmoid(m)
    ids_ref[...] = ids
    onehot = (ids == lane).astype(jnp.float32)
    r = lax.broadcasted_iota(jnp.int32, (TB, TB), 0)
    c = lax.broadcasted_iota(jnp.int32, (TB, TB), 1)
    tri = (r >= c).astype(jnp.float32)
    csum = jnp.dot(tri, onehot, preferred_element_type=jnp.float32)

    @pl.when(pid == 0)
    def _():
        acc[...] = jnp.zeros_like(acc)

    base = acc[...]
    rank = jnp.sum(onehot * (csum - 1.0 + base), axis=1, keepdims=True)
    rank_ref[...] = rank.astype(jnp.int32)
    total = base + csum[TB - 1:TB, :]
    acc[...] = total
    cnt_ref[...] = total.astype(jnp.int32)


def _router(x, Wr):
    return pl.pallas_call(
        _router_body,
        grid=(S // TB,),
        in_specs=[
            pl.BlockSpec((TB, D), lambda i: (i, 0)),
            pl.BlockSpec((D, E), lambda i: (0, 0)),
        ],
        out_specs=[
            pl.BlockSpec((TB, D), lambda i: (i, 0)),
            pl.BlockSpec((TB, 1), lambda i: (i, 0)),
            pl.BlockSpec((TB, 1), lambda i: (i, 0)),
            pl.BlockSpec((1, E), lambda i: (0, 0)),
        ],
        out_shape=[
            jax.ShapeDtypeStruct((S, D), jnp.float32),
            jax.ShapeDtypeStruct((S, 1), jnp.int32),
            jax.ShapeDtypeStruct((S, 1), jnp.int32),
            jax.ShapeDtypeStruct((1, E), jnp.int32),
        ],
        scratch_shapes=[pltpu.VMEM((1, E), jnp.float32)],
    )(x, Wr)


def _meta_body(ids_ref, rank_ref, cnt_ref, pos_ref, brow_ref, bexp_ref):
    counts = cnt_ref[...].astype(jnp.float32)
    padded = jnp.ceil(counts / BM) * BM
    rr = lax.broadcasted_iota(jnp.int32, (E, E), 0)
    cc = lax.broadcasted_iota(jnp.int32, (E, E), 1)
    strict_upper = (rr < cc).astype(jnp.float32)
    offs = jnp.dot(padded, strict_upper,
                   preferred_element_type=jnp.float32)
    total = jnp.sum(padded)

    ids = ids_ref[...]
    lane = lax.broadcasted_iota(jnp.int32, (S, E), 1)
    onehot = ids == lane
    myoff = jnp.sum(jnp.where(onehot, jnp.broadcast_to(offs, (S, E)), 0.0),
                    axis=1, keepdims=True)
    pos_ref[...] = myoff.astype(jnp.int32) + rank_ref[...]

    gi = lax.broadcasted_iota(jnp.int32, (G, 1), 0)
    used = (total / BM).astype(jnp.int32)
    brow = jnp.minimum(gi, used - 1)
    brow_ref[...] = brow
    rowstart = (brow * BM).astype(jnp.float32)
    cmp = jnp.broadcast_to(offs, (G, E)) <= rowstart
    bexp = jnp.sum(cmp.astype(jnp.int32), axis=1, keepdims=True) - 1
    bexp_ref[...] = jnp.clip(bexp, 0, E - 1)


def _meta(ids, rank, counts):
    return pl.pallas_call(
        _meta_body,
        out_shape=[
            jax.ShapeDtypeStruct((S, 1), jnp.int32),
            jax.ShapeDtypeStruct((G, 1), jnp.int32),
            jax.ShapeDtypeStruct((G, 1), jnp.int32),
        ],
    )(ids, rank, counts)


IB = 512


def _invert_body(pos_ref, sidx_ref):
    rb = pl.program_id(0)
    pos = pos_ref[...]
    slot = lax.broadcasted_iota(jnp.int32, (S, IB), 1) + rb * IB
    tok = lax.broadcasted_iota(jnp.int32, (S, IB), 0)
    hit = pos == slot
    si = jnp.sum(jnp.where(hit, tok, 0), axis=0, keepdims=True)
    nhit = jnp.sum(hit.astype(jnp.int32), axis=0, keepdims=True)
    fill = (lax.broadcasted_iota(jnp.int32, (1, IB), 1) + rb * IB) % S
    sidx_ref[...] = jnp.where(nhit > 0, si, fill).reshape(1, 1, IB)


def _invert(pos):
    return pl.pallas_call(
        _invert_body,
        grid=(R // IB,),
        in_specs=[pl.BlockSpec((S, 1), lambda i: (0, 0))],
        out_specs=pl.BlockSpec((1, 1, IB), lambda i: (i, 0, 0)),
        out_shape=jax.ShapeDtypeStruct((R // IB, 1, IB), jnp.int32),
    )(pos)


def _dispatch(sidx, moe_x):
    rows_per_tile = R // NTILES
    chunk = 64
    mesh = plsc.VectorSubcoreMesh(core_axis_name="c", subcore_axis_name="s")

    @functools.partial(
        pl.kernel, mesh=mesh,
        out_type=jax.ShapeDtypeStruct((R, D), jnp.float32),
        scratch_types=[
            pltpu.VMEM((chunk,), jnp.int32),
            pltpu.VMEM((chunk, D), jnp.float32),
            pltpu.SemaphoreType.DMA,
        ])
    def k(sidx_hbm, mx_hbm, xs_hbm, cidx_v, rows_v, sem):
        wid = lax.axis_index("s") * 2 + lax.axis_index("c")
        base = wid * rows_per_tile
        for c in range(rows_per_tile // chunk):
            off = base + c * chunk
            pltpu.sync_copy(sidx_hbm.at[pl.ds(off, chunk)], cidx_v)
            pltpu.async_copy(mx_hbm.at[cidx_v], rows_v, sem).wait()
            pltpu.sync_copy(rows_v, xs_hbm.at[pl.ds(off, chunk)])

    return k(sidx, moe_x)


def _ffn_body(brow_ref, bexp_ref, x_ref, wgu_ref, wdn_ref, out_ref):
    del brow_ref, bexp_ref
    xb = x_ref[...]
    gu = jnp.dot(xb, wgu_ref[0], preferred_element_type=jnp.float32)
    g = gu[:, :FF]
    u = gu[:, FF:]
    act = g * jax.nn.sigmoid(g) * u
    out_ref[...] = jnp.dot(act, wdn_ref[0], preferred_element_type=jnp.float32)


def _ffn(x_sorted, Wgu_e, Wdn_e, brow, bexp):
    grid_spec = pltpu.PrefetchScalarGridSpec(
        num_scalar_prefetch=2,
        grid=(G,),
        in_specs=[
            pl.BlockSpec((BM, D), lambda g, brow, bexp: (brow[g], 0)),
            pl.BlockSpec((1, D, 2 * FF), lambda g, brow, bexp: (bexp[g], 0, 0)),
            pl.BlockSpec((1, FF, D), lambda g, brow, bexp: (bexp[g], 0, 0)),
        ],
        out_specs=pl.BlockSpec((BM, D), lambda g, brow, bexp: (brow[g], 0)),
    )
    return pl.pallas_call(
        _ffn_body,
        grid_spec=grid_spec,
        out_shape=jax.ShapeDtypeStruct((R, D), jnp.float32),
    )(brow, bexp, x_sorted, Wgu_e, Wdn_e)


def _unsort(pos, out_sorted):
    rows_per_tile = S // NTILES
    mesh = plsc.VectorSubcoreMesh(core_axis_name="c", subcore_axis_name="s")

    @functools.partial(
        pl.kernel, mesh=mesh,
        out_type=jax.ShapeDtypeStruct((S, D), jnp.float32),
        scratch_types=[
            pltpu.VMEM((rows_per_tile,), jnp.int32),
            pltpu.VMEM((rows_per_tile, D), jnp.float32),
            pltpu.SemaphoreType.DMA,
        ])
    def k(pos_hbm, os_hbm, om_hbm, idx_v, rows_v, sem):
        wid = lax.axis_index("s") * 2 + lax.axis_index("c")
        base = wid * rows_per_tile
        pltpu.sync_copy(pos_hbm.at[pl.ds(base, rows_per_tile)], idx_v)
        pltpu.async_copy(os_hbm.at[idx_v], rows_v, sem).wait()
        pltpu.sync_copy(rows_v, om_hbm.at[pl.ds(base, rows_per_tile)])

    return k(pos, out_sorted)


def _shared_body(x_ref, wgu_ref, wdn_ref, moe_ref, out_ref):
    x = x_ref[...]
    gu = jnp.dot(x, wgu_ref[...], preferred_element_type=jnp.float32)
    g = gu[:, :FF]
    u = gu[:, FF:]
    act = g * jax.nn.sigmoid(g) * u
    out_ref[...] = (jnp.dot(act, wdn_ref[...], preferred_element_type=jnp.float32)
                    + moe_ref[...])


def _shared(x, Wgu_s, Wdn_s, out_moe):
    return pl.pallas_call(
        _shared_body,
        grid=(S // TB,),
        in_specs=[
            pl.BlockSpec((TB, D), lambda i: (i, 0)),
            pl.BlockSpec((D, 2 * FF), lambda i: (0, 0)),
            pl.BlockSpec((FF, D), lambda i: (0, 0)),
            pl.BlockSpec((TB, D), lambda i: (i, 0)),
        ],
        out_specs=pl.BlockSpec((TB, D), lambda i: (i, 0)),
        out_shape=jax.ShapeDtypeStruct((S, D), jnp.float32),
    )(x, Wgu_s, Wdn_s, out_moe)


def kernel(hidden_states, Wr, Wgu_e, Wdn_e, Wgu_s, Wdn_s):
    b, s, d = hidden_states.shape
    x = hidden_states.reshape(s, d)
    moe_x, ids, rank, counts = _router(x, Wr)
    pos2, brow2, bexp2 = _meta(ids, rank, counts)
    pos = pos2.reshape(s)
    brow = brow2.reshape(G)
    bexp = bexp2.reshape(G)
    sidx = _invert(pos2).reshape(R)
    x_sorted = _dispatch(sidx, moe_x)
    out_sorted = _ffn(x_sorted, Wgu_e, Wdn_e, brow, bexp)
    out_moe = _unsort(pos, out_sorted)
    out = _shared(x, Wgu_s, Wdn_s, out_moe)
    return out.reshape(b, s, d)


# speedup vs baseline: 1.1673x; 1.1673x over previous
import functools

import jax
import jax.numpy as jnp
from jax import lax
from jax.experimental import pallas as pl
from jax.experimental.pallas import tpu as pltpu
from jax.experimental.pallas import tpu_sc as plsc

S = 2048
D = 1024
FF = 512
E = 64
BM = 64
R = 6144
G = 96
TB = 256
NTILES = 32


def _router_body(x_ref, wr_ref, moe_ref, ids_ref, rank_ref, cnt_ref, acc):
    pid = pl.program_id(0)
    x = x_ref[...]
    logits = jnp.dot(x, wr_ref[...], preferred_element_type=jnp.float32)
    m = jnp.max(logits, axis=1, keepdims=True)
    lane = lax.broadcasted_iota(jnp.int32, (TB, E), 1)
    ids = jnp.min(jnp.where(logits == m, lane, E), axis=1, keepdims=True)
    moe_ref[...] = x * jax.nn.sigmoid(m)
    ids_ref[...] = ids
    onehot = (ids == lane).astype(jnp.float32)
    r = lax.broadcasted_iota(jnp.int32, (TB, TB), 0)
    c = lax.broadcasted_iota(jnp.int32, (TB, TB), 1)
    tri = (r >= c).astype(jnp.float32)
    csum = jnp.dot(tri, onehot, preferred_element_type=jnp.float32)

    @pl.when(pid == 0)
    def _():
        acc[...] = jnp.zeros_like(acc)

    base = acc[...]
    rank = jnp.sum(onehot * (csum - 1.0 + base), axis=1, keepdims=True)
    rank_ref[...] = rank.astype(jnp.int32)
    total = base + csum[TB - 1:TB, :]
    acc[...] = total
    cnt_ref[...] = total.astype(jnp.int32)


def _router(x, Wr):
    return pl.pallas_call(
        _router_body,
        grid=(S // TB,),
        in_specs=[
            pl.BlockSpec((TB, D), lambda i: (i, 0)),
            pl.BlockSpec((D, E), lambda i: (0, 0)),
        ],
        out_specs=[
            pl.BlockSpec((TB, D), lambda i: (i, 0)),
            pl.BlockSpec((TB, 1), lambda i: (i, 0)),
            pl.BlockSpec((TB, 1), lambda i: (i, 0)),
            pl.BlockSpec((1, E), lambda i: (0, 0)),
        ],
        out_shape=[
            jax.ShapeDtypeStruct((S, D), jnp.float32),
            jax.ShapeDtypeStruct((S, 1), jnp.int32),
            jax.ShapeDtypeStruct((S, 1), jnp.int32),
            jax.ShapeDtypeStruct((1, E), jnp.int32),
        ],
        scratch_shapes=[pltpu.VMEM((1, E), jnp.float32)],
    )(x, Wr)


def _meta_body(ids_ref, rank_ref, cnt_ref, pos_ref, brow_ref, bexp_ref):
    counts = cnt_ref[...].astype(jnp.float32)
    padded = jnp.ceil(counts / BM) * BM
    rr = lax.broadcasted_iota(jnp.int32, (E, E), 0)
    cc = lax.broadcasted_iota(jnp.int32, (E, E), 1)
    strict_upper = (rr < cc).astype(jnp.float32)
    offs = jnp.dot(padded, strict_upper,
                   preferred_element_type=jnp.float32)
    total = jnp.sum(padded)

    ids = ids_ref[...]
    lane = lax.broadcasted_iota(jnp.int32, (S, E), 1)
    onehot = ids == lane
    myoff = jnp.sum(jnp.where(onehot, jnp.broadcast_to(offs, (S, E)), 0.0),
                    axis=1, keepdims=True)
    pos_ref[...] = myoff.astype(jnp.int32) + rank_ref[...]

    gi = lax.broadcasted_iota(jnp.int32, (G, 1), 0)
    used = (total / BM).astype(jnp.int32)
    brow = jnp.minimum(gi, used - 1)
    brow_ref[...] = brow
    rowstart = (brow * BM).astype(jnp.float32)
    cmp = jnp.broadcast_to(offs, (G, E)) <= rowstart
    bexp = jnp.sum(cmp.astype(jnp.int32), axis=1, keepdims=True) - 1
    bexp_ref[...] = jnp.clip(bexp, 0, E - 1)


def _meta(ids, rank, counts):
    return pl.pallas_call(
        _meta_body,
        out_shape=[
            jax.ShapeDtypeStruct((S, 1), jnp.int32),
            jax.ShapeDtypeStruct((G, 1), jnp.int32),
            jax.ShapeDtypeStruct((G, 1), jnp.int32),
        ],
    )(ids, rank, counts)


def _dispatch(pos, moe_x):
    rows_per_tile = S // NTILES
    mesh = plsc.VectorSubcoreMesh(core_axis_name="c", subcore_axis_name="s")

    @functools.partial(
        pl.kernel, mesh=mesh,
        out_type=jax.ShapeDtypeStruct((R, D), jnp.float32),
        scratch_types=[
            pltpu.VMEM((rows_per_tile,), jnp.int32),
            pltpu.VMEM((rows_per_tile, D), jnp.float32),
            pltpu.SemaphoreType.DMA,
        ])
    def k(pos_hbm, mx_hbm, xs_hbm, cidx_v, rows_v, sem):
        wid = lax.axis_index("s") * 2 + lax.axis_index("c")
        base = wid * rows_per_tile
        pltpu.sync_copy(pos_hbm.at[pl.ds(base, rows_per_tile)], cidx_v)
        pltpu.sync_copy(mx_hbm.at[pl.ds(base, rows_per_tile)], rows_v)
        pltpu.async_copy(rows_v, xs_hbm.at[cidx_v], sem).wait()

    return k(pos, moe_x)


def _ffn_body(brow_ref, bexp_ref, x_ref, wgu_ref, wdn_ref, out_ref):
    del brow_ref, bexp_ref
    xb = x_ref[...]
    gu = jnp.dot(xb, wgu_ref[0], preferred_element_type=jnp.float32)
    g = gu[:, :FF]
    u = gu[:, FF:]
    act = g * jax.nn.sigmoid(g) * u
    out_ref[...] = jnp.dot(act, wdn_ref[0], preferred_element_type=jnp.float32)


def _ffn(x_sorted, Wgu_e, Wdn_e, brow, bexp):
    grid_spec = pltpu.PrefetchScalarGridSpec(
        num_scalar_prefetch=2,
        grid=(G,),
        in_specs=[
            pl.BlockSpec((BM, D), lambda g, brow, bexp: (brow[g], 0)),
            pl.BlockSpec((1, D, 2 * FF), lambda g, brow, bexp: (bexp[g], 0, 0)),
            pl.BlockSpec((1, FF, D), lambda g, brow, bexp: (bexp[g], 0, 0)),
        ],
        out_specs=pl.BlockSpec((BM, D), lambda g, brow, bexp: (brow[g], 0)),
    )
    return pl.pallas_call(
        _ffn_body,
        grid_spec=grid_spec,
        out_shape=jax.ShapeDtypeStruct((R, D), jnp.float32),
    )(brow, bexp, x_sorted, Wgu_e, Wdn_e)


def _unsort(pos, out_sorted):
    rows_per_tile = S // NTILES
    mesh = plsc.VectorSubcoreMesh(core_axis_name="c", subcore_axis_name="s")

    @functools.partial(
        pl.kernel, mesh=mesh,
        out_type=jax.ShapeDtypeStruct((S, D), jnp.float32),
        scratch_types=[
            pltpu.VMEM((rows_per_tile,), jnp.int32),
            pltpu.VMEM((rows_per_tile, D), jnp.float32),
            pltpu.SemaphoreType.DMA,
        ])
    def k(pos_hbm, os_hbm, om_hbm, idx_v, rows_v, sem):
        wid = lax.axis_index("s") * 2 + lax.axis_index("c")
        base = wid * rows_per_tile
        pltpu.sync_copy(pos_hbm.at[pl.ds(base, rows_per_tile)], idx_v)
        pltpu.async_copy(os_hbm.at[idx_v], rows_v, sem).wait()
        pltpu.sync_copy(rows_v, om_hbm.at[pl.ds(base, rows_per_tile)])

    return k(pos, out_sorted)


def _shared_body(x_ref, wgu_ref, wdn_ref, moe_ref, out_ref):
    x = x_ref[...]
    gu = jnp.dot(x, wgu_ref[...], preferred_element_type=jnp.float32)
    g = gu[:, :FF]
    u = gu[:, FF:]
    act = g * jax.nn.sigmoid(g) * u
    out_ref[...] = (jnp.dot(act, wdn_ref[...], preferred_element_type=jnp.float32)
                    + moe_ref[...])


def _shared(x, Wgu_s, Wdn_s, out_moe):
    return pl.pallas_call(
        _shared_body,
        grid=(S // TB,),
        in_specs=[
            pl.BlockSpec((TB, D), lambda i: (i, 0)),
            pl.BlockSpec((D, 2 * FF), lambda i: (0, 0)),
            pl.BlockSpec((FF, D), lambda i: (0, 0)),
            pl.BlockSpec((TB, D), lambda i: (i, 0)),
        ],
        out_specs=pl.BlockSpec((TB, D), lambda i: (i, 0)),
        out_shape=jax.ShapeDtypeStruct((S, D), jnp.float32),
    )(x, Wgu_s, Wdn_s, out_moe)


def kernel(hidden_states, Wr, Wgu_e, Wdn_e, Wgu_s, Wdn_s):
    b, s, d = hidden_states.shape
    x = hidden_states.reshape(s, d)
    moe_x, ids, rank, counts = _router(x, Wr)
    pos2, brow2, bexp2 = _meta(ids, rank, counts)
    pos = pos2.reshape(s)
    brow = brow2.reshape(G)
    bexp = bexp2.reshape(G)
    x_sorted = _dispatch(pos, moe_x)
    out_sorted = _ffn(x_sorted, Wgu_e, Wdn_e, brow, bexp)
    out_moe = _unsort(pos, out_sorted)
    out = _shared(x, Wgu_s, Wdn_s, out_moe)
    return out.reshape(b, s, d)


# speedup vs baseline: 1.3107x; 1.1228x over previous
import functools

import jax
import jax.numpy as jnp
from jax import lax
from jax.experimental import pallas as pl
from jax.experimental.pallas import tpu as pltpu
from jax.experimental.pallas import tpu_sc as plsc

S = 2048
D = 1024
FF = 512
E = 64
BM = 64
R = 6144
G = 96
TB = 256
NTILES = 32


def _router_body(x_ref, wr_ref, moe_ref, ids_ref, rank_ref, cnt_ref, acc):
    pid = pl.program_id(0)
    x = x_ref[...]
    logits = jnp.dot(x, wr_ref[...], preferred_element_type=jnp.float32)
    m = jnp.max(logits, axis=1, keepdims=True)
    lane = lax.broadcasted_iota(jnp.int32, (TB, E), 1)
    ids = jnp.min(jnp.where(logits == m, lane, E), axis=1, keepdims=True)
    moe_ref[...] = x * jax.nn.sigmoid(m)
    ids_ref[...] = ids
    onehot = (ids == lane).astype(jnp.float32)
    r = lax.broadcasted_iota(jnp.int32, (TB, TB), 0)
    c = lax.broadcasted_iota(jnp.int32, (TB, TB), 1)
    tri = (r >= c).astype(jnp.float32)
    csum = jnp.dot(tri, onehot, preferred_element_type=jnp.float32)

    @pl.when(pid == 0)
    def _():
        acc[...] = jnp.zeros_like(acc)

    base = acc[...]
    rank = jnp.sum(onehot * (csum - 1.0 + base), axis=1, keepdims=True)
    rank_ref[...] = rank.astype(jnp.int32)
    total = base + csum[TB - 1:TB, :]
    acc[...] = total
    cnt_ref[...] = total.astype(jnp.int32)


def _router(x, Wr):
    return pl.pallas_call(
        _router_body,
        grid=(S // TB,),
        in_specs=[
            pl.BlockSpec((TB, D), lambda i: (i, 0)),
            pl.BlockSpec((D, E), lambda i: (0, 0)),
        ],
        out_specs=[
            pl.BlockSpec((TB, D), lambda i: (i, 0)),
            pl.BlockSpec((TB, 1), lambda i: (i, 0)),
            pl.BlockSpec((TB, 1), lambda i: (i, 0)),
            pl.BlockSpec((1, E), lambda i: (0, 0)),
        ],
        out_shape=[
            jax.ShapeDtypeStruct((S, D), jnp.float32),
            jax.ShapeDtypeStruct((S, 1), jnp.int32),
            jax.ShapeDtypeStruct((S, 1), jnp.int32),
            jax.ShapeDtypeStruct((1, E), jnp.int32),
        ],
        scratch_shapes=[pltpu.VMEM((1, E), jnp.float32)],
    )(x, Wr)


def _meta_body(ids_ref, rank_ref, cnt_ref, pos_ref, brow_ref, bexp_ref):
    counts = cnt_ref[...].astype(jnp.float32)
    padded = jnp.ceil(counts / BM) * BM
    rr = lax.broadcasted_iota(jnp.int32, (E, E), 0)
    cc = lax.broadcasted_iota(jnp.int32, (E, E), 1)
    strict_upper = (rr < cc).astype(jnp.float32)
    offs = jnp.dot(padded, strict_upper,
                   preferred_element_type=jnp.float32)
    total = jnp.sum(padded)

    ids = ids_ref[...]
    lane = lax.broadcasted_iota(jnp.int32, (S, E), 1)
    onehot = ids == lane
    myoff = jnp.sum(jnp.where(onehot, jnp.broadcast_to(offs, (S, E)), 0.0),
                    axis=1, keepdims=True)
    pos_ref[...] = myoff.astype(jnp.int32) + rank_ref[...]

    gi = lax.broadcasted_iota(jnp.int32, (G, 1), 0)
    used = (total / BM).astype(jnp.int32)
    brow = jnp.minimum(gi, used - 1)
    brow_ref[...] = brow
    rowstart = (brow * BM).astype(jnp.float32)
    cmp = jnp.broadcast_to(offs, (G, E)) <= rowstart
    bexp = jnp.sum(cmp.astype(jnp.int32), axis=1, keepdims=True) - 1
    bexp_ref[...] = jnp.clip(bexp, 0, E - 1)


def _meta(ids, rank, counts):
    return pl.pallas_call(
        _meta_body,
        out_shape=[
            jax.ShapeDtypeStruct((S, 1), jnp.int32),
            jax.ShapeDtypeStruct((G, 1), jnp.int32),
            jax.ShapeDtypeStruct((G, 1), jnp.int32),
        ],
    )(ids, rank, counts)


def _dispatch(pos, moe_x):
    rows_per_tile = S // NTILES
    mesh = plsc.VectorSubcoreMesh(core_axis_name="c", subcore_axis_name="s")

    @functools.partial(
        pl.kernel, mesh=mesh,
        out_type=jax.ShapeDtypeStruct((R, D), jnp.float32),
        scratch_types=[
            pltpu.VMEM((rows_per_tile,), jnp.int32),
            pltpu.VMEM((rows_per_tile, D), jnp.float32),
            pltpu.SemaphoreType.DMA,
        ])
    def k(pos_hbm, mx_hbm, xs_hbm, cidx_v, rows_v, sem):
        wid = lax.axis_index("s") * 2 + lax.axis_index("c")
        base = wid * rows_per_tile
        pltpu.sync_copy(pos_hbm.at[pl.ds(base, rows_per_tile)], cidx_v)
        pltpu.sync_copy(mx_hbm.at[pl.ds(base, rows_per_tile)], rows_v)
        pltpu.async_copy(rows_v, xs_hbm.at[cidx_v], sem).wait()

    return k(pos, moe_x)


def _ffn_body(brow_ref, bexp_ref, x_ref, wgu_ref, wdn_ref, out_ref):
    del bexp_ref
    gi = pl.program_id(0)

    @pl.when(brow_ref[gi] == gi)
    def _():
        xb = x_ref[...]
        gu = jnp.dot(xb, wgu_ref[0], preferred_element_type=jnp.float32)
        g = gu[:, :FF]
        u = gu[:, FF:]
        act = g * jax.nn.sigmoid(g) * u
        out_ref[...] = jnp.dot(act, wdn_ref[0],
                               preferred_element_type=jnp.float32)


def _ffn(x_sorted, Wgu_e, Wdn_e, brow, bexp):
    grid_spec = pltpu.PrefetchScalarGridSpec(
        num_scalar_prefetch=2,
        grid=(G,),
        in_specs=[
            pl.BlockSpec((BM, D), lambda g, brow, bexp: (brow[g], 0)),
            pl.BlockSpec((1, D, 2 * FF), lambda g, brow, bexp: (bexp[g], 0, 0)),
            pl.BlockSpec((1, FF, D), lambda g, brow, bexp: (bexp[g], 0, 0)),
        ],
        out_specs=pl.BlockSpec((BM, D), lambda g, brow, bexp: (brow[g], 0)),
    )
    return pl.pallas_call(
        _ffn_body,
        grid_spec=grid_spec,
        out_shape=jax.ShapeDtypeStruct((R, D), jnp.float32),
    )(brow, bexp, x_sorted, Wgu_e, Wdn_e)


def _unsort(pos, out_sorted):
    rows_per_tile = S // NTILES
    mesh = plsc.VectorSubcoreMesh(core_axis_name="c", subcore_axis_name="s")

    @functools.partial(
        pl.kernel, mesh=mesh,
        out_type=jax.ShapeDtypeStruct((S, D), jnp.float32),
        scratch_types=[
            pltpu.VMEM((rows_per_tile,), jnp.int32),
            pltpu.VMEM((rows_per_tile, D), jnp.float32),
            pltpu.SemaphoreType.DMA,
        ])
    def k(pos_hbm, os_hbm, om_hbm, idx_v, rows_v, sem):
        wid = lax.axis_index("s") * 2 + lax.axis_index("c")
        base = wid * rows_per_tile
        pltpu.sync_copy(pos_hbm.at[pl.ds(base, rows_per_tile)], idx_v)
        pltpu.async_copy(os_hbm.at[idx_v], rows_v, sem).wait()
        pltpu.sync_copy(rows_v, om_hbm.at[pl.ds(base, rows_per_tile)])

    return k(pos, out_sorted)


def _shared_body(x_ref, wgu_ref, wdn_ref, moe_ref, out_ref):
    x = x_ref[...]
    gu = jnp.dot(x, wgu_ref[...], preferred_element_type=jnp.float32)
    g = gu[:, :FF]
    u = gu[:, FF:]
    act = g * jax.nn.sigmoid(g) * u
    out_ref[...] = (jnp.dot(act, wdn_ref[...], preferred_element_type=jnp.float32)
                    + moe_ref[...])


def _shared(x, Wgu_s, Wdn_s, out_moe):
    return pl.pallas_call(
        _shared_body,
        grid=(S // TB,),
        in_specs=[
            pl.BlockSpec((TB, D), lambda i: (i, 0)),
            pl.BlockSpec((D, 2 * FF), lambda i: (0, 0)),
            pl.BlockSpec((FF, D), lambda i: (0, 0)),
            pl.BlockSpec((TB, D), lambda i: (i, 0)),
        ],
        out_specs=pl.BlockSpec((TB, D), lambda i: (i, 0)),
        out_shape=jax.ShapeDtypeStruct((S, D), jnp.float32),
    )(x, Wgu_s, Wdn_s, out_moe)


def kernel(hidden_states, Wr, Wgu_e, Wdn_e, Wgu_s, Wdn_s):
    b, s, d = hidden_states.shape
    x = hidden_states.reshape(s, d)
    moe_x, ids, rank, counts = _router(x, Wr)
    pos2, brow2, bexp2 = _meta(ids, rank, counts)
    pos = pos2.reshape(s)
    brow = brow2.reshape(G)
    bexp = bexp2.reshape(G)
    x_sorted = _dispatch(pos, moe_x)
    out_sorted = _ffn(x_sorted, Wgu_e, Wdn_e, brow, bexp)
    out_moe = _unsort(pos, out_sorted)
    out = _shared(x, Wgu_s, Wdn_s, out_moe)
    return out.reshape(b, s, d)
